# baseline (speedup 1.0000x reference)
import jax
import jax.numpy as jnp
from jax.experimental import pallas as pl
from jax.experimental.pallas import tpu as pltpu

_IOU_THRESHOLD = 0.8
_K_SELECT = 1024
_N = 20000
_NB = 20
_BR = 8
_COLS = 128
_BLK = _BR * _COLS
_ROWS = _NB * _BR
_N_PAD = _ROWS * _COLS
_NEG_INF = float("-inf")


def _combine(a, b):
    cond = (a[0] > b[0]) | ((a[0] == b[0]) & (a[1] < b[1]))
    return tuple(jnp.where(cond, x, y) for x, y in zip(a, b))


def _nms_body(x1_ref, y1_ref, x2_ref, y2_ref, sc_ref, out_ref):
    lane1 = jax.lax.broadcasted_iota(jnp.int32, (1, _COLS), 1)
    f = (jax.lax.broadcasted_iota(jnp.int32, (_BR, _COLS), 0) * _COLS
         + jax.lax.broadcasted_iota(jnp.int32, (_BR, _COLS), 1))

    def blk(ref, b):
        return ref[pl.ds(_BR * b, _BR), :]

    def body(i, state):
        (pP, pX1, pY1, pX2, pY2), masked = state
        parea = (pX2 - pX1) * (pY2 - pY1)
        acc = None
        new_masked = []
        for b in range(_NB):
            x1b = blk(x1_ref, b)
            y1b = blk(y1_ref, b)
            x2b = blk(x2_ref, b)
            y2b = blk(y2_ref, b)
            areas_b = (x2b - x1b) * (y2b - y1b)
            xx1 = jnp.maximum(pX1, x1b)
            yy1 = jnp.maximum(pY1, y1b)
            xx2 = jnp.minimum(pX2, x2b)
            yy2 = jnp.minimum(pY2, y2b)
            inter = jnp.maximum(xx2 - xx1, 0.0) * jnp.maximum(yy2 - yy1, 0.0)
            iou = inter / (parea + areas_b - inter + 1e-8)
            kill = (iou > _IOU_THRESHOLD) | ((f + _BLK * b) == pP)
            mb = jnp.where(kill, _NEG_INF, masked[b])
            new_masked.append(mb)
            tb = (mb, f + _BLK * b, x1b, y1b, x2b, y2b, blk(sc_ref, b))
            acc = tb if acc is None else _combine(acc, tb)
        for ax, sh in ((0, 4), (0, 2), (0, 1), (1, 64), (1, 32), (1, 16),
                       (1, 8), (1, 4), (1, 2), (1, 1)):
            acc = _combine(acc, tuple(jnp.roll(x, sh, axis=ax) for x in acc))
        _, mP, mX1, mY1, mX2, mY2, mS = acc

        row = (jnp.where(lane1 == 0, mX1[0:1, :], 0.0)
               + jnp.where(lane1 == 1, mY1[0:1, :], 0.0)
               + jnp.where(lane1 == 2, mX2[0:1, :], 0.0)
               + jnp.where(lane1 == 3, mY2[0:1, :], 0.0)
               + jnp.where(lane1 == 4, mS[0:1, :], 0.0))
        out_ref[pl.ds(i, 1), :] = row
        return ((mP, mX1, mY1, mX2, mY2), tuple(new_masked))

    far = jnp.full((_BR, _COLS), -1.0e30, jnp.float32)
    prev0 = (jnp.full((_BR, _COLS), -1, jnp.int32), far, far, far, far)
    masked0 = tuple(blk(sc_ref, b) for b in range(_NB))
    jax.lax.fori_loop(0, _K_SELECT, body, (prev0, masked0))


def kernel(boxes, scores):
    pad = _N_PAD - _N
    x1 = jnp.pad(boxes[:, 0], (0, pad)).reshape(_ROWS, _COLS)
    y1 = jnp.pad(boxes[:, 1], (0, pad)).reshape(_ROWS, _COLS)
    x2 = jnp.pad(boxes[:, 2], (0, pad)).reshape(_ROWS, _COLS)
    y2 = jnp.pad(boxes[:, 3], (0, pad)).reshape(_ROWS, _COLS)
    sc = jnp.pad(scores, (0, pad), constant_values=_NEG_INF).reshape(_ROWS, _COLS)

    out = pl.pallas_call(
        _nms_body,
        out_shape=jax.ShapeDtypeStruct((_K_SELECT, _COLS), jnp.float32),

    )(x1, y1, x2, y2, sc)
    return out[:, :5]


# speedup vs baseline: 1.0104x; 1.0104x over previous
import jax
import jax.numpy as jnp
from jax.experimental import pallas as pl
from jax.experimental.pallas import tpu as pltpu

_IOU_THRESHOLD = 0.8
_K_SELECT = 1024
_N = 20000
_NB = 20
_BR = 8
_COLS = 128
_BLK = _BR * _COLS
_ROWS = _NB * _BR
_N_PAD = _ROWS * _COLS
_NEG_INF = float("-inf")


def _combine(a, b):
    cond = (a[0] > b[0]) | ((a[0] == b[0]) & (a[1] < b[1]))
    return tuple(jnp.where(cond, x, y) for x, y in zip(a, b))


def _nms_body(x1_ref, y1_ref, x2_ref, y2_ref, sc_ref, out_ref):
    lane1 = jax.lax.broadcasted_iota(jnp.int32, (1, _COLS), 1)
    f = (jax.lax.broadcasted_iota(jnp.int32, (_BR, _COLS), 0) * _COLS
         + jax.lax.broadcasted_iota(jnp.int32, (_BR, _COLS), 1))

    def blk(ref, b):
        return ref[pl.ds(_BR * b, _BR), :]

    def body(i, state):
        (pP, pX1, pY1, pX2, pY2), masked = state
        parea = (pX2 - pX1) * (pY2 - pY1)
        accs = [None, None, None, None]
        new_masked = []
        for b in range(_NB):
            x1b = blk(x1_ref, b)
            y1b = blk(y1_ref, b)
            x2b = blk(x2_ref, b)
            y2b = blk(y2_ref, b)
            areas_b = (x2b - x1b) * (y2b - y1b)
            xx1 = jnp.maximum(pX1, x1b)
            yy1 = jnp.maximum(pY1, y1b)
            xx2 = jnp.minimum(pX2, x2b)
            yy2 = jnp.minimum(pY2, y2b)
            inter = jnp.maximum(xx2 - xx1, 0.0) * jnp.maximum(yy2 - yy1, 0.0)
            iou = inter / (parea + areas_b - inter + 1e-8)
            kill = (iou > _IOU_THRESHOLD) | ((f + _BLK * b) == pP)
            mb = jnp.where(kill, _NEG_INF, masked[b])
            new_masked.append(mb)
            tb = (mb, f + _BLK * b, x1b, y1b, x2b, y2b, blk(sc_ref, b))
            g = b // 5
            accs[g] = tb if accs[g] is None else _combine(accs[g], tb)
        acc = _combine(_combine(accs[0], accs[1]),
                       _combine(accs[2], accs[3]))
        for ax, sh in ((0, 4), (0, 2), (0, 1), (1, 64), (1, 32), (1, 16),
                       (1, 8), (1, 4), (1, 2), (1, 1)):
            acc = _combine(acc, tuple(jnp.roll(x, sh, axis=ax) for x in acc))
        _, mP, mX1, mY1, mX2, mY2, mS = acc

        row = (jnp.where(lane1 == 0, mX1[0:1, :], 0.0)
               + jnp.where(lane1 == 1, mY1[0:1, :], 0.0)
               + jnp.where(lane1 == 2, mX2[0:1, :], 0.0)
               + jnp.where(lane1 == 3, mY2[0:1, :], 0.0)
               + jnp.where(lane1 == 4, mS[0:1, :], 0.0))
        out_ref[pl.ds(i, 1), :] = row
        return ((mP, mX1, mY1, mX2, mY2), tuple(new_masked))

    far = jnp.full((_BR, _COLS), -1.0e30, jnp.float32)
    prev0 = (jnp.full((_BR, _COLS), -1, jnp.int32), far, far, far, far)
    masked0 = tuple(blk(sc_ref, b) for b in range(_NB))
    jax.lax.fori_loop(0, _K_SELECT, body, (prev0, masked0))


def kernel(boxes, scores):
    pad = _N_PAD - _N
    x1 = jnp.pad(boxes[:, 0], (0, pad)).reshape(_ROWS, _COLS)
    y1 = jnp.pad(boxes[:, 1], (0, pad)).reshape(_ROWS, _COLS)
    x2 = jnp.pad(boxes[:, 2], (0, pad)).reshape(_ROWS, _COLS)
    y2 = jnp.pad(boxes[:, 3], (0, pad)).reshape(_ROWS, _COLS)
    sc = jnp.pad(scores, (0, pad), constant_values=_NEG_INF).reshape(_ROWS, _COLS)

    out = pl.pallas_call(
        _nms_body,
        out_shape=jax.ShapeDtypeStruct((_K_SELECT, _COLS), jnp.float32),

    )(x1, y1, x2, y2, sc)
    return out[:, :5]


# speedup vs baseline: 1.0827x; 1.0715x over previous
import jax
import jax.numpy as jnp
from jax.experimental import pallas as pl

_IOU_THRESHOLD = 0.8
_K_SELECT = 1024
_N = 20000
_NB = 20
_BR = 8
_COLS = 128
_BLK = _BR * _COLS
_ROWS = _NB * _BR
_N_PAD = _ROWS * _COLS
_NEG_INF = float("-inf")


def _nms_body(x1_ref, y1_ref, x2_ref, y2_ref, sc_ref, out_ref):
    lane1 = jax.lax.broadcasted_iota(jnp.int32, (1, _COLS), 1)
    f = (jax.lax.broadcasted_iota(jnp.int32, (_BR, _COLS), 0) * _COLS
         + jax.lax.broadcasted_iota(jnp.int32, (_BR, _COLS), 1))

    def blk(ref, b):
        return ref[pl.ds(_BR * b, _BR), :]

    def body(i, state):
        (pP, pX1, pY1, pX2, pY2), masked = state
        parea = (pX2 - pX1) * (pY2 - pY1)
        new_masked = []
        for b in range(_NB):
            x1b = blk(x1_ref, b)
            y1b = blk(y1_ref, b)
            x2b = blk(x2_ref, b)
            y2b = blk(y2_ref, b)
            areas_b = (x2b - x1b) * (y2b - y1b)
            xx1 = jnp.maximum(pX1, x1b)
            yy1 = jnp.maximum(pY1, y1b)
            xx2 = jnp.minimum(pX2, x2b)
            yy2 = jnp.minimum(pY2, y2b)
            inter = jnp.maximum(xx2 - xx1, 0.0) * jnp.maximum(yy2 - yy1, 0.0)
            iou = inter / (parea + areas_b - inter + 1e-8)
            kill = (iou > _IOU_THRESHOLD) | ((f + _BLK * b) == pP)
            new_masked.append(jnp.where(kill, _NEG_INF, masked[b]))

        def treemax(xs):
            while len(xs) > 1:
                nxt = [jnp.maximum(xs[j], xs[j + 1])
                       for j in range(0, len(xs) - 1, 2)]
                if len(xs) % 2:
                    nxt.append(xs[-1])
                xs = nxt
            return xs[0]

        def treemin(xs):
            while len(xs) > 1:
                nxt = [jnp.minimum(xs[j], xs[j + 1])
                       for j in range(0, len(xs) - 1, 2)]
                if len(xs) % 2:
                    nxt.append(xs[-1])
                xs = nxt
            return xs[0]

        m = jnp.max(treemax(new_masked))
        idx = jnp.min(treemin([jnp.where(new_masked[b] == m, f + _BLK * b,
                                         _N_PAD) for b in range(_NB)]))

        def extract(ref):
            return jnp.max(treemax(
                [jnp.where((f + _BLK * b) == idx, blk(ref, b), _NEG_INF)
                 for b in range(_NB)]))

        x1i = extract(x1_ref)
        y1i = extract(y1_ref)
        x2i = extract(x2_ref)
        y2i = extract(y2_ref)
        si = extract(sc_ref)

        row = (jnp.where(lane1 == 0, x1i, 0.0)
               + jnp.where(lane1 == 1, y1i, 0.0)
               + jnp.where(lane1 == 2, x2i, 0.0)
               + jnp.where(lane1 == 3, y2i, 0.0)
               + jnp.where(lane1 == 4, si, 0.0))
        out_ref[pl.ds(i, 1), :] = row

        shape = (_BR, _COLS)
        prev = (jnp.full(shape, idx, jnp.int32), jnp.full(shape, x1i),
                jnp.full(shape, y1i), jnp.full(shape, x2i),
                jnp.full(shape, y2i))
        return (prev, tuple(new_masked))

    far = jnp.full((_BR, _COLS), -1.0e30, jnp.float32)
    prev0 = (jnp.full((_BR, _COLS), -1, jnp.int32), far, far, far, far)
    masked0 = tuple(blk(sc_ref, b) for b in range(_NB))
    jax.lax.fori_loop(0, _K_SELECT, body, (prev0, masked0))


def kernel(boxes, scores):
    pad = _N_PAD - _N
    x1 = jnp.pad(boxes[:, 0], (0, pad)).reshape(_ROWS, _COLS)
    y1 = jnp.pad(boxes[:, 1], (0, pad)).reshape(_ROWS, _COLS)
    x2 = jnp.pad(boxes[:, 2], (0, pad)).reshape(_ROWS, _COLS)
    y2 = jnp.pad(boxes[:, 3], (0, pad)).reshape(_ROWS, _COLS)
    sc = jnp.pad(scores, (0, pad), constant_values=_NEG_INF).reshape(_ROWS, _COLS)

    out = pl.pallas_call(
        _nms_body,
        out_shape=jax.ShapeDtypeStruct((_K_SELECT, _COLS), jnp.float32),
    )(x1, y1, x2, y2, sc)
    return out[:, :5]


# speedup vs baseline: 1.1047x; 1.0204x over previous
import jax
import jax.numpy as jnp
from jax.experimental import pallas as pl

_IOU_THRESHOLD = 0.8
_K_SELECT = 1024
_N = 20000
_NB = 20
_BR = 8
_COLS = 128
_BLK = _BR * _COLS
_ROWS = _NB * _BR
_N_PAD = _ROWS * _COLS
_NEG_INF = float("-inf")


def _nms_body(x1_ref, y1_ref, x2_ref, y2_ref, sc_ref, out_ref):
    lane1 = jax.lax.broadcasted_iota(jnp.int32, (1, _COLS), 1)
    f = (jax.lax.broadcasted_iota(jnp.int32, (_BR, _COLS), 0) * _COLS
         + jax.lax.broadcasted_iota(jnp.int32, (_BR, _COLS), 1))

    def blk(ref, b):
        return ref[pl.ds(_BR * b, _BR), :]

    def body(i, state):
        (pP, pX1, pY1, pX2, pY2), masked = state
        parea = (pX2 - pX1) * (pY2 - pY1)
        new_masked = []
        for b in range(_NB):
            x1b = blk(x1_ref, b)
            y1b = blk(y1_ref, b)
            x2b = blk(x2_ref, b)
            y2b = blk(y2_ref, b)
            areas_b = (x2b - x1b) * (y2b - y1b)
            xx1 = jnp.maximum(pX1, x1b)
            yy1 = jnp.maximum(pY1, y1b)
            xx2 = jnp.minimum(pX2, x2b)
            yy2 = jnp.minimum(pY2, y2b)
            inter = jnp.maximum(xx2 - xx1, 0.0) * jnp.maximum(yy2 - yy1, 0.0)
            iou = inter / (parea + areas_b - inter + 1e-8)
            kill = (iou > _IOU_THRESHOLD) | ((f + _BLK * b) == pP)
            new_masked.append(jnp.where(kill, _NEG_INF, masked[b]))

        def treemax(xs):
            while len(xs) > 1:
                nxt = [jnp.maximum(xs[j], xs[j + 1])
                       for j in range(0, len(xs) - 1, 2)]
                if len(xs) % 2:
                    nxt.append(xs[-1])
                xs = nxt
            return xs[0]

        def treemin(xs):
            while len(xs) > 1:
                nxt = [jnp.minimum(xs[j], xs[j + 1])
                       for j in range(0, len(xs) - 1, 2)]
                if len(xs) % 2:
                    nxt.append(xs[-1])
                xs = nxt
            return xs[0]

        idx = jnp.argmax(
            jnp.concatenate(new_masked, axis=0).reshape(-1)).astype(jnp.int32)

        def extract(ref):
            return jnp.max(treemax(
                [jnp.where((f + _BLK * b) == idx, blk(ref, b), _NEG_INF)
                 for b in range(_NB)]))

        x1i = extract(x1_ref)
        y1i = extract(y1_ref)
        x2i = extract(x2_ref)
        y2i = extract(y2_ref)
        si = extract(sc_ref)

        row = (jnp.where(lane1 == 0, x1i, 0.0)
               + jnp.where(lane1 == 1, y1i, 0.0)
               + jnp.where(lane1 == 2, x2i, 0.0)
               + jnp.where(lane1 == 3, y2i, 0.0)
               + jnp.where(lane1 == 4, si, 0.0))
        out_ref[pl.ds(i, 1), :] = row

        shape = (_BR, _COLS)
        prev = (jnp.full(shape, idx, jnp.int32), jnp.full(shape, x1i),
                jnp.full(shape, y1i), jnp.full(shape, x2i),
                jnp.full(shape, y2i))
        return (prev, tuple(new_masked))

    far = jnp.full((_BR, _COLS), -1.0e30, jnp.float32)
    prev0 = (jnp.full((_BR, _COLS), -1, jnp.int32), far, far, far, far)
    masked0 = tuple(blk(sc_ref, b) for b in range(_NB))
    jax.lax.fori_loop(0, _K_SELECT, body, (prev0, masked0))


def kernel(boxes, scores):
    pad = _N_PAD - _N
    x1 = jnp.pad(boxes[:, 0], (0, pad)).reshape(_ROWS, _COLS)
    y1 = jnp.pad(boxes[:, 1], (0, pad)).reshape(_ROWS, _COLS)
    x2 = jnp.pad(boxes[:, 2], (0, pad)).reshape(_ROWS, _COLS)
    y2 = jnp.pad(boxes[:, 3], (0, pad)).reshape(_ROWS, _COLS)
    sc = jnp.pad(scores, (0, pad), constant_values=_NEG_INF).reshape(_ROWS, _COLS)

    out = pl.pallas_call(
        _nms_body,
        out_shape=jax.ShapeDtypeStruct((_K_SELECT, _COLS), jnp.float32),
    )(x1, y1, x2, y2, sc)
    return out[:, :5]
